# baseline (speedup 1.0000x reference)
import jax
import jax.numpy as jnp
import numpy as np
from jax import lax
from jax.experimental import pallas as pl
from jax.experimental.pallas import tpu as pltpu
from jax.experimental.pallas import tpu_sc as plsc

NC, NS = 2, 16
NW = NC * NS
B, H, D, NL = 4096, 200, 64, 128
BPW = B // NW
C0, C1 = 104, 96


def _sc_pool_body(idx_hbm, emb_hbm, out_hbm, idx_v, rows_a, rows_b, acc_v,
                  sem_a, sem_b):
    c = lax.axis_index("c")
    s = lax.axis_index("s")
    wid = s * NC + c
    base = wid * BPW
    pltpu.sync_copy(idx_hbm.at[pl.ds(base, BPW)], idx_v)

    def start(r, rows_ref, sem):
        pltpu.async_copy(
            emb_hbm.at[idx_v.at[r, pl.ds(0, C0)]], rows_ref.at[pl.ds(0, C0)], sem)
        pltpu.async_copy(
            emb_hbm.at[idx_v.at[r, pl.ds(C0, C1)]], rows_ref.at[pl.ds(C0, C1)], sem)

    def wait(r, rows_ref, sem):
        pltpu.make_async_copy(
            emb_hbm.at[idx_v.at[r, pl.ds(0, C0)]], rows_ref.at[pl.ds(0, C0)], sem).wait()
        pltpu.make_async_copy(
            emb_hbm.at[idx_v.at[r, pl.ds(C0, C1)]], rows_ref.at[pl.ds(C0, C1)], sem).wait()

    def reduce(rows_ref, r):
        def jbody(j, accs):
            new = []
            for k in range(2):
                w = rows_ref[j, pl.ds(16 * k, 16)]
                lo = plsc.bitcast(w << 16, jnp.float32)
                hi = plsc.bitcast(w & jnp.int32(-65536), jnp.float32)
                new.append(accs[2 * k] + lo)
                new.append(accs[2 * k + 1] + hi)
            return tuple(new)
        accs = lax.fori_loop(
            0, H, jbody,
            tuple(jnp.zeros((16,), jnp.float32) for _ in range(4)),
            unroll=8)
        for pos, k in enumerate((0, 2, 1, 3)):
            acc_v[r, pl.ds(16 * pos, 16)] = accs[k]

    start(0, rows_a, sem_a)

    def body2(k, carry):
        r = 2 * k
        start(r + 1, rows_b, sem_b)
        wait(r, rows_a, sem_a)
        reduce(rows_a, r)

        @pl.when(k < BPW // 2 - 1)
        def _():
            start(r + 2, rows_a, sem_a)

        wait(r + 1, rows_b, sem_b)
        reduce(rows_b, r + 1)
        return carry

    lax.fori_loop(0, BPW // 2, body2, 0)
    pltpu.sync_copy(acc_v, out_hbm.at[pl.ds(base, BPW)])


def _sc_pool(input_ids, emb):
    mesh = plsc.VectorSubcoreMesh(
        core_axis_name="c", subcore_axis_name="s",
        num_cores=NC, num_subcores=NS)
    return pl.kernel(
        _sc_pool_body,
        out_type=jax.ShapeDtypeStruct((B, D), jnp.float32),
        mesh=mesh,
        scratch_types=[
            pltpu.VMEM((BPW, H), jnp.int32),
            pltpu.VMEM((H, D // 2), jnp.int32),
            pltpu.VMEM((H, D // 2), jnp.int32),
            pltpu.VMEM((BPW, D), jnp.float32),
            pltpu.SemaphoreType.DMA,
            pltpu.SemaphoreType.DMA,
        ],
        compiler_params=pltpu.CompilerParams(
            use_tc_tiling_on_sc=False, needs_layout_passes=False),
    )(input_ids, emb)


VB = 8192
G = VB // 4
V = 1000000
GSH = G.bit_length() - 1


def _pack_rows(x):
    a = jax.lax.bitcast_convert_type(x[0:32, :], jnp.int32)
    bb = jax.lax.bitcast_convert_type(x[32:64, :], jnp.int32)
    ra = (a + 0x7FFF + ((a >> 16) & 1)) >> 16
    rb = (bb + 0x7FFF + ((bb >> 16) & 1)) >> 16
    return (rb << 16) | (ra & 0xFFFF)


def _tc_transpose_body(x_ref, o_ref):
    w = _pack_rows(x_ref[...])
    o_ref[...] = jnp.concatenate(
        [w[:, g * G:(g + 1) * G].T for g in range(4)], axis=1)


NMAIN = (V // VB) * VB
NTAIL = V - NMAIN
HT = NTAIL // 4


def _tc_tail_body(x_ref, tbl_ref, o_ref):
    del tbl_ref
    w = _pack_rows(x_ref[...])
    for g in range(4):
        o_ref[0:HT, 32 * g:32 * g + 32] = w[:, g * HT:(g + 1) * HT].T


def _tc_transpose(emt):
    nblk = NMAIN // VB
    main = pl.pallas_call(
        _tc_transpose_body,
        grid=(nblk,),
        in_specs=[pl.BlockSpec((64, VB), lambda i: (0, i))],
        out_specs=pl.BlockSpec((G, 128), lambda i: (i, 0)),
        out_shape=jax.ShapeDtypeStruct((V // 4, 128), jnp.int32),
    )(emt)
    return pl.pallas_call(
        _tc_tail_body,
        grid=(1,),
        in_specs=[pl.BlockSpec((64, NTAIL), lambda i: (0, 0)),
                  pl.BlockSpec(memory_space=pl.ANY)],
        out_specs=pl.BlockSpec((G, 128), lambda i: (NMAIN // VB, 0)),
        out_shape=jax.ShapeDtypeStruct((V // 4, 128), jnp.int32),
        input_output_aliases={1: 0},
    )(emt[:, NMAIN:], main)


def _tc_remap_body(i_ref, o_ref):
    v = i_ref[...]
    j = v & (VB - 1)
    main_row = (v - j) + 4 * (j & (G - 1)) + (j >> GSH)
    jt = v - NMAIN
    tail_row = NMAIN + 4 * (jt % HT) + jt // HT
    o_ref[...] = jnp.where(v >= NMAIN, tail_row, main_row)


def _tc_remap(input_ids):
    return pl.pallas_call(
        _tc_remap_body,
        out_shape=jax.ShapeDtypeStruct((B, H), jnp.int32),
    )(input_ids)


def _tc_matmul_body(x_ref, wt_ref, b_ref, o_ref):
    x = x_ref[...] * (1.0 / H)
    o_ref[...] = jnp.dot(x, wt_ref[...],
                         preferred_element_type=jnp.float32) + b_ref[...]


def _tc_matmul(x, wt, b2):
    return pl.pallas_call(
        _tc_matmul_body,
        out_shape=jax.ShapeDtypeStruct((B, NL), jnp.float32),
    )(x, wt, b2)


def kernel(input, emb, W, b):
    ids2 = _tc_remap(input)
    lin = _tc_transpose(emb.T).reshape(V, 32)
    pooled = _sc_pool(ids2, lin)
    return _tc_matmul(pooled, W.T, b.reshape(1, NL))


# speedup vs baseline: 1.0364x; 1.0364x over previous
import jax
import jax.numpy as jnp
import numpy as np
from jax import lax
from jax.experimental import pallas as pl
from jax.experimental.pallas import tpu as pltpu
from jax.experimental.pallas import tpu_sc as plsc

NC, NS = 2, 16
NW = NC * NS
B, H, D, NL = 4096, 200, 64, 128
BPW = B // NW
C0, C1 = 104, 96


def _sc_pool_body(idx_hbm, emb_hbm, out_hbm, idx_v, rows_a, rows_b, acc_v,
                  sem_a, sem_b):
    c = lax.axis_index("c")
    s = lax.axis_index("s")
    wid = s * NC + c
    base = wid * BPW
    pltpu.sync_copy(idx_hbm.at[pl.ds(base, BPW)], idx_v)

    def start(r, rows_ref, sem):
        pltpu.async_copy(
            emb_hbm.at[idx_v.at[r, pl.ds(0, C0)]], rows_ref.at[pl.ds(0, C0)], sem)
        pltpu.async_copy(
            emb_hbm.at[idx_v.at[r, pl.ds(C0, C1)]], rows_ref.at[pl.ds(C0, C1)], sem)

    def wait(r, rows_ref, sem):
        pltpu.make_async_copy(
            emb_hbm.at[idx_v.at[r, pl.ds(0, C0)]], rows_ref.at[pl.ds(0, C0)], sem).wait()
        pltpu.make_async_copy(
            emb_hbm.at[idx_v.at[r, pl.ds(C0, C1)]], rows_ref.at[pl.ds(C0, C1)], sem).wait()

    def reduce(rows_ref, r):
        def jbody(j, accs):
            new = []
            for k in range(2):
                w = rows_ref[j, pl.ds(16 * k, 16)]
                lo = plsc.bitcast(w << 16, jnp.float32)
                hi = plsc.bitcast(w & jnp.int32(-65536), jnp.float32)
                new.append(accs[2 * k] + lo)
                new.append(accs[2 * k + 1] + hi)
            return tuple(new)
        accs = lax.fori_loop(
            0, H, jbody,
            tuple(jnp.zeros((16,), jnp.float32) for _ in range(4)),
            unroll=8)
        for pos, k in enumerate((0, 2, 1, 3)):
            acc_v[r, pl.ds(16 * pos, 16)] = accs[k]

    start(0, rows_a, sem_a)

    def body2(k, carry):
        r = 2 * k
        start(r + 1, rows_b, sem_b)
        wait(r, rows_a, sem_a)
        reduce(rows_a, r)

        @pl.when(k < BPW // 2 - 1)
        def _():
            start(r + 2, rows_a, sem_a)

        wait(r + 1, rows_b, sem_b)
        reduce(rows_b, r + 1)
        return carry

    lax.fori_loop(0, BPW // 2, body2, 0)
    pltpu.sync_copy(acc_v, out_hbm.at[pl.ds(base, BPW)])


def _sc_pool(input_ids, emb):
    mesh = plsc.VectorSubcoreMesh(
        core_axis_name="c", subcore_axis_name="s",
        num_cores=NC, num_subcores=NS)
    return pl.kernel(
        _sc_pool_body,
        out_type=jax.ShapeDtypeStruct((B, D), jnp.float32),
        mesh=mesh,
        scratch_types=[
            pltpu.VMEM((BPW, H), jnp.int32),
            pltpu.VMEM((H, D // 2), jnp.int32),
            pltpu.VMEM((H, D // 2), jnp.int32),
            pltpu.VMEM((BPW, D), jnp.float32),
            pltpu.SemaphoreType.DMA,
            pltpu.SemaphoreType.DMA,
        ],
        compiler_params=pltpu.CompilerParams(
            use_tc_tiling_on_sc=False, needs_layout_passes=False),
    )(input_ids, emb)


VB = 16384
G = VB // 4
V = 1000000
GSH = G.bit_length() - 1


def _pack_rows(x):
    a = jax.lax.bitcast_convert_type(x[0:32, :], jnp.int32)
    bb = jax.lax.bitcast_convert_type(x[32:64, :], jnp.int32)
    ra = (a + 0x7FFF + ((a >> 16) & 1)) >> 16
    rb = (bb + 0x7FFF + ((bb >> 16) & 1)) >> 16
    return (rb << 16) | (ra & 0xFFFF)


def _tc_transpose_body(x_ref, o_ref):
    w = _pack_rows(x_ref[...])
    o_ref[...] = jnp.concatenate(
        [w[:, g * G:(g + 1) * G].T for g in range(4)], axis=1)


NMAIN = (V // VB) * VB
NTAIL = V - NMAIN
HT = NTAIL // 4


def _tc_tail_body(x_ref, tbl_ref, o_ref):
    del tbl_ref
    w = _pack_rows(x_ref[...])
    for g in range(4):
        o_ref[0:HT, 32 * g:32 * g + 32] = w[:, g * HT:(g + 1) * HT].T


def _tc_transpose(emt):
    nblk = NMAIN // VB
    main = pl.pallas_call(
        _tc_transpose_body,
        grid=(nblk,),
        in_specs=[pl.BlockSpec((64, VB), lambda i: (0, i))],
        out_specs=pl.BlockSpec((G, 128), lambda i: (i, 0)),
        out_shape=jax.ShapeDtypeStruct((V // 4, 128), jnp.int32),
    )(emt)
    return pl.pallas_call(
        _tc_tail_body,
        grid=(1,),
        in_specs=[pl.BlockSpec((64, NTAIL), lambda i: (0, 0)),
                  pl.BlockSpec(memory_space=pl.ANY)],
        out_specs=pl.BlockSpec((G, 128), lambda i: (NMAIN // VB, 0)),
        out_shape=jax.ShapeDtypeStruct((V // 4, 128), jnp.int32),
        input_output_aliases={1: 0},
    )(emt[:, NMAIN:], main)


def _tc_remap_body(i_ref, o_ref):
    v = i_ref[...]
    j = v & (VB - 1)
    main_row = (v - j) + 4 * (j & (G - 1)) + (j >> GSH)
    jt = v - NMAIN
    tail_row = NMAIN + 4 * (jt % HT) + jt // HT
    o_ref[...] = jnp.where(v >= NMAIN, tail_row, main_row)


def _tc_remap(input_ids):
    return pl.pallas_call(
        _tc_remap_body,
        out_shape=jax.ShapeDtypeStruct((B, H), jnp.int32),
    )(input_ids)


def _tc_matmul_body(x_ref, wt_ref, b_ref, o_ref):
    x = x_ref[...] * (1.0 / H)
    o_ref[...] = jnp.dot(x, wt_ref[...],
                         preferred_element_type=jnp.float32) + b_ref[...]


def _tc_matmul(x, wt, b2):
    return pl.pallas_call(
        _tc_matmul_body,
        out_shape=jax.ShapeDtypeStruct((B, NL), jnp.float32),
    )(x, wt, b2)


def kernel(input, emb, W, b):
    ids2 = _tc_remap(input)
    lin = _tc_transpose(emb.T).reshape(V, 32)
    pooled = _sc_pool(ids2, lin)
    return _tc_matmul(pooled, W.T, b.reshape(1, NL))
